# Initial kernel scaffold; baseline (speedup 1.0000x reference)
import functools

import jax
import jax.numpy as jnp
from jax import lax
from jax.experimental import pallas as pl
from jax.experimental.pallas import tpu as pltpu
from jax.experimental.pallas import tpu_sc as plsc

EMBED_DIM = 32
BATCH = 4096
HIST = 200
B_TOT = BATCH * HIST
NUM_WORKERS = 32
B_PER_W = B_TOT // NUM_WORKERS
CHUNK = 1024
N_CHUNKS = B_PER_W // CHUNK


def _sc_gather(tags_flat, table):
    mesh = plsc.VectorSubcoreMesh(core_axis_name="c", subcore_axis_name="s")

    @functools.partial(
        pl.kernel,
        mesh=mesh,
        out_type=jax.ShapeDtypeStruct((B_TOT, EMBED_DIM), jnp.float32),
        scratch_types=[
            pltpu.VMEM((CHUNK,), jnp.int32),
            pltpu.VMEM((CHUNK, EMBED_DIM), jnp.float32),
            pltpu.SemaphoreType.DMA,
        ],
    )
    def k(tags_hbm, table_hbm, out_hbm, idx_v, rows_v, sem):
        wid = lax.axis_index("s") * 2 + lax.axis_index("c")
        base = wid * B_PER_W

        def body(i, carry):
            off = base + i * CHUNK
            pltpu.sync_copy(tags_hbm.at[pl.ds(off, CHUNK)], idx_v)
            pltpu.async_copy(table_hbm.at[idx_v], rows_v, sem).wait()
            pltpu.sync_copy(rows_v, out_hbm.at[pl.ds(off, CHUNK)])
            return carry

        lax.fori_loop(0, N_CHUNKS, body, 0)

    return k(tags_flat, table)


def kernel(tags, table):
    tags_flat = tags.reshape(-1).astype(jnp.int32)
    out = _sc_gather(tags_flat, table)
    return out.reshape(BATCH, HIST, EMBED_DIM)


# speedup vs baseline: 1.4580x; 1.4580x over previous
import functools

import jax
import jax.numpy as jnp
from jax import lax
from jax.experimental import pallas as pl
from jax.experimental.pallas import tpu as pltpu
from jax.experimental.pallas import tpu_sc as plsc

EMBED_DIM = 32
BATCH = 4096
HIST = 200
B_TOT = BATCH * HIST
NUM_WORKERS = 32
B_PER_W = B_TOT // NUM_WORKERS
CHUNK = 1024
N_CHUNKS = B_PER_W // CHUNK


def _sc_gather(tags_flat, table):
    mesh = plsc.VectorSubcoreMesh(core_axis_name="c", subcore_axis_name="s")

    @functools.partial(
        pl.kernel,
        mesh=mesh,
        out_type=jax.ShapeDtypeStruct((B_TOT, EMBED_DIM), jnp.float32),
        scratch_types=[
            pltpu.VMEM((CHUNK,), jnp.int32),
            pltpu.VMEM((CHUNK, EMBED_DIM), jnp.float32),
            pltpu.SemaphoreType.DMA,
        ],
        compiler_params=pltpu.CompilerParams(use_tc_tiling_on_sc=False),
    )
    def k(tags_hbm, table_hbm, out_hbm, idx_v, rows_v, sem):
        wid = lax.axis_index("s") * 2 + lax.axis_index("c")
        base = wid * B_PER_W

        def body(i, carry):
            off = base + i * CHUNK
            pltpu.sync_copy(tags_hbm.at[pl.ds(off, CHUNK)], idx_v)
            pltpu.async_copy(table_hbm.at[idx_v], rows_v, sem).wait()
            pltpu.sync_copy(rows_v, out_hbm.at[pl.ds(off, CHUNK)])
            return carry

        lax.fori_loop(0, N_CHUNKS, body, 0)

    return k(tags_flat, table)


def kernel(tags, table):
    tags_flat = tags.reshape(-1).astype(jnp.int32)
    out = _sc_gather(tags_flat, table)
    return out.reshape(BATCH, HIST, EMBED_DIM)


# speedup vs baseline: 1.4988x; 1.0279x over previous
import functools

import jax
import jax.numpy as jnp
from jax import lax
from jax.experimental import pallas as pl
from jax.experimental.pallas import tpu as pltpu
from jax.experimental.pallas import tpu_sc as plsc

EMBED_DIM = 32
BATCH = 4096
HIST = 200
B_TOT = BATCH * HIST
NUM_WORKERS = 32
B_PER_W = B_TOT // NUM_WORKERS
CHUNK = 640
N_CHUNKS = B_PER_W // CHUNK
NBUF = 4


def _sc_gather(tags_flat, table):
    mesh = plsc.VectorSubcoreMesh(core_axis_name="c", subcore_axis_name="s")

    @functools.partial(
        pl.kernel,
        mesh=mesh,
        out_type=jax.ShapeDtypeStruct((B_TOT, EMBED_DIM), jnp.float32),
        scratch_types=[
            pltpu.VMEM((B_PER_W,), jnp.int32),
            [pltpu.VMEM((CHUNK, EMBED_DIM), jnp.float32) for _ in range(NBUF)],
            [pltpu.SemaphoreType.DMA for _ in range(NBUF)],
        ],
        compiler_params=pltpu.CompilerParams(use_tc_tiling_on_sc=False),
    )
    def k(tags_hbm, table_hbm, out_hbm, idx_v, rows, sems):
        wid = lax.axis_index("s") * 2 + lax.axis_index("c")
        base = wid * B_PER_W

        pltpu.sync_copy(tags_hbm.at[pl.ds(base, B_PER_W)], idx_v)

        def start_gather(c, b):
            pltpu.async_copy(
                table_hbm.at[idx_v.at[pl.ds(c * CHUNK, CHUNK)]], rows[b], sems[b]
            )

        for b in range(NBUF):
            start_gather(b, b)

        @pl.loop(0, N_CHUNKS, step=NBUF)
        def _(g):
            for b in range(NBUF):
                c = g + b
                pltpu.make_async_copy(
                    table_hbm.at[idx_v.at[pl.ds(c * CHUNK, CHUNK)]], rows[b], sems[b]
                ).wait()
                pltpu.sync_copy(rows[b], out_hbm.at[pl.ds(base + c * CHUNK, CHUNK)])

                @pl.when(c + NBUF < N_CHUNKS)
                def _():
                    start_gather(c + NBUF, b)

    return k(tags_flat, table)


def kernel(tags, table):
    tags_flat = tags.reshape(-1).astype(jnp.int32)
    out = _sc_gather(tags_flat, table)
    return out.reshape(BATCH, HIST, EMBED_DIM)


# speedup vs baseline: 1.4992x; 1.0003x over previous
import functools

import jax
import jax.numpy as jnp
from jax import lax
from jax.experimental import pallas as pl
from jax.experimental.pallas import tpu as pltpu
from jax.experimental.pallas import tpu_sc as plsc

EMBED_DIM = 32
BATCH = 4096
HIST = 200
B_TOT = BATCH * HIST
NUM_WORKERS = 32
B_PER_W = BATCH // NUM_WORKERS
CHUNK = HIST
N_CHUNKS = B_PER_W
NBUF = 4


def _sc_gather(tags_flat, table):
    mesh = plsc.VectorSubcoreMesh(core_axis_name="c", subcore_axis_name="s")

    @functools.partial(
        pl.kernel,
        mesh=mesh,
        out_type=jax.ShapeDtypeStruct((BATCH, HIST, EMBED_DIM), jnp.float32),
        scratch_types=[
            pltpu.VMEM((B_PER_W * HIST,), jnp.int32),
            [pltpu.VMEM((HIST, EMBED_DIM), jnp.float32) for _ in range(NBUF)],
            [pltpu.SemaphoreType.DMA for _ in range(NBUF)],
        ],
        compiler_params=pltpu.CompilerParams(use_tc_tiling_on_sc=False),
    )
    def k(tags_hbm, table_hbm, out_hbm, idx_v, rows, sems):
        wid = lax.axis_index("s") * 2 + lax.axis_index("c")
        base_b = wid * B_PER_W

        pltpu.sync_copy(tags_hbm.at[pl.ds(base_b * HIST, B_PER_W * HIST)], idx_v)

        def start_gather(c, b):
            pltpu.async_copy(
                table_hbm.at[idx_v.at[pl.ds(c * CHUNK, CHUNK)]], rows[b], sems[b]
            )

        for b in range(NBUF):
            start_gather(b, b)

        @pl.loop(0, N_CHUNKS, step=NBUF)
        def _(g):
            for b in range(NBUF):
                c = g + b
                pltpu.make_async_copy(
                    table_hbm.at[idx_v.at[pl.ds(c * CHUNK, CHUNK)]], rows[b], sems[b]
                ).wait()
                pltpu.sync_copy(rows[b], out_hbm.at[base_b + c])

                @pl.when(c + NBUF < N_CHUNKS)
                def _():
                    start_gather(c + NBUF, b)

    return k(tags_flat, table)


def kernel(tags, table):
    tags_flat = tags.reshape(-1).astype(jnp.int32)
    return _sc_gather(tags_flat, table)


# speedup vs baseline: 2.0494x; 1.3669x over previous
import functools

import jax
import jax.numpy as jnp
from jax import lax
from jax.experimental import pallas as pl
from jax.experimental.pallas import tpu as pltpu
from jax.experimental.pallas import tpu_sc as plsc

EMBED_DIM = 32
BATCH = 4096
HIST = 200
B_TOT = BATCH * HIST
NUM_WORKERS = 32
B_PER_W = BATCH // NUM_WORKERS
CHUNK = HIST
N_CHUNKS = B_PER_W
NBUF = 4


def _sc_gather(tags_flat, table):
    mesh = plsc.VectorSubcoreMesh(core_axis_name="c", subcore_axis_name="s")

    @functools.partial(
        pl.kernel,
        mesh=mesh,
        out_type=jax.ShapeDtypeStruct((BATCH, HIST, 128), jnp.float32),
        scratch_types=[
            pltpu.VMEM((B_PER_W * HIST,), jnp.int32),
            [pltpu.VMEM((HIST, EMBED_DIM), jnp.float32) for _ in range(NBUF)],
            [pltpu.SemaphoreType.DMA for _ in range(NBUF)],
        ],
        compiler_params=pltpu.CompilerParams(use_tc_tiling_on_sc=False),
    )
    def k(tags_hbm, table_hbm, out_hbm, idx_v, rows, sems):
        wid = lax.axis_index("s") * 2 + lax.axis_index("c")
        base_b = wid * B_PER_W

        pltpu.sync_copy(tags_hbm.at[pl.ds(base_b * HIST, B_PER_W * HIST)], idx_v)

        def start_gather(c, b):
            pltpu.async_copy(
                table_hbm.at[idx_v.at[pl.ds(c * CHUNK, CHUNK)]], rows[b], sems[b]
            )

        for b in range(NBUF):
            start_gather(b, b)

        @pl.loop(0, N_CHUNKS, step=NBUF)
        def _(g):
            for b in range(NBUF):
                c = g + b
                pltpu.make_async_copy(
                    table_hbm.at[idx_v.at[pl.ds(c * CHUNK, CHUNK)]], rows[b], sems[b]
                ).wait()
                pltpu.sync_copy(
                    rows[b], out_hbm.at[base_b + c, :, pl.ds(0, EMBED_DIM)]
                )

                @pl.when(c + NBUF < N_CHUNKS)
                def _():
                    start_gather(c + NBUF, b)

    return k(tags_flat, table)


def kernel(tags, table):
    tags_flat = tags.reshape(-1).astype(jnp.int32)
    out_padded = _sc_gather(tags_flat, table)
    return lax.slice(out_padded, (0, 0, 0), (BATCH, HIST, EMBED_DIM))
